# baseline (speedup 1.0000x reference)
import jax
import jax.numpy as jnp
from jax.experimental import pallas as pl
from jax.experimental.pallas import tpu as pltpu

NUM_CLASSES = 21
ROW_F = NUM_CLASSES + 4
R = 4000
NBLK = 80
GAMMA_IS_2 = True
BETA = 0.5
NEG_POS_RATIO = 3.0


def _stage_a(yp_ref, yb_ref, anc_ref, alpha_ref, neg_ref, stats_ref, acc_ref):
    i = pl.program_id(0)

    @pl.when(i == 0)
    def _init():
        acc_ref[0] = 0.0
        acc_ref[1] = 0.0
        acc_ref[2] = 0.0
        acc_ref[3] = 0.0

    c_pre = yp_ref[:, :NUM_CLASSES]
    c_hat = yb_ref[:, :NUM_CLASSES]
    bp = yp_ref[:, NUM_CLASSES:]
    bh = yb_ref[:, NUM_CLASSES:]
    anc = anc_ref[...]

    m = jnp.max(c_pre, axis=1, keepdims=True)
    se = jnp.sum(jnp.exp(c_pre - m), axis=1, keepdims=True)
    tl = jnp.sum(c_hat * c_pre, axis=1, keepdims=True)
    at = jnp.sum(c_hat * alpha_ref[0:1, :NUM_CLASSES], axis=1, keepdims=True)
    logpt = tl - m - jnp.log(se)
    pt = jnp.exp(logpt)
    omp = 1.0 - pt
    loss = -(omp * omp) * (logpt * at)

    col0 = c_hat[:, 0:1]
    negm = col0 > 0.5
    posm = jnp.logical_not(negm)

    acc_ref[0] = acc_ref[0] + jnp.sum(jnp.where(posm, loss, 0.0))
    acc_ref[2] = acc_ref[2] + jnp.sum(jnp.where(posm, 1.0, 0.0))
    acc_ref[3] = acc_ref[3] + jnp.sum(jnp.where(negm, 1.0, 0.0))

    neg_ref[...] = jnp.where(negm, loss + 0.0, -1.0)

    a_xy = anc[:, 0:2]
    wh_a = anc[:, 2:4] - a_xy
    c_a = a_xy + 0.5 * wh_a
    dxy_p = c_a + bp[:, 0:2] * wh_a
    dwh_p = wh_a * jnp.exp(bp[:, 2:4])
    lt_p = dxy_p - 0.5 * dwh_p
    rb_p = dxy_p + 0.5 * dwh_p
    dxy_t = c_a + bh[:, 0:2] * wh_a
    dwh_t = wh_a * jnp.exp(bh[:, 2:4])
    lt_t = dxy_t - 0.5 * dwh_t
    rb_t = dxy_t + 0.5 * dwh_t
    lt = jnp.maximum(lt_p, lt_t)
    rb = jnp.minimum(rb_p, rb_t)
    whc = jnp.maximum(rb - lt, 0.0)
    inter = whc[:, 0:1] * whc[:, 1:2]
    wh1 = jnp.maximum(rb_p - lt_p, 0.0)
    area1 = wh1[:, 0:1] * wh1[:, 1:2]
    wh2 = jnp.maximum(rb_t - lt_t, 0.0)
    area2 = wh2[:, 0:1] * wh2[:, 1:2]
    union = area1 + area2 - inter
    iou = inter / (union + 1e-8)
    acc_ref[1] = acc_ref[1] + jnp.sum(jnp.where(posm, 1.0 - iou, 0.0))

    @pl.when(i == NBLK - 1)
    def _fin():
        lane = jax.lax.broadcasted_iota(jnp.int32, (8, 128), 1)
        v = jnp.where(lane == 0, acc_ref[0],
                      jnp.where(lane == 1, acc_ref[1],
                                jnp.where(lane == 2, acc_ref[2], acc_ref[3])))
        stats_ref[...] = v


def _stage_b(neg_ref, stats_ref, out_ref):
    lane = jax.lax.broadcasted_iota(jnp.int32, (8, 128), 1)
    row = jax.lax.broadcasted_iota(jnp.int32, (8, 128), 0)
    stats = stats_ref[...]
    sel = (row == 0)

    def pick(j):
        return jnp.sum(jnp.where(sel & (lane == j), stats, 0.0))

    pos_sum = pick(0)
    reg_sum = pick(1)
    npos = pick(2)
    nneg = pick(3)
    k = jnp.minimum(nneg, NEG_POS_RATIO * npos)

    vals = neg_ref[...]
    bits = jax.lax.bitcast_convert_type(vals, jnp.int32)

    def body(_, carry):
        lo, hi = carry
        mid = lo + (hi - lo) // 2
        cnt = jnp.sum(jnp.where(bits >= mid, 1.0, 0.0))
        ok = cnt >= k
        return (jnp.where(ok, mid, lo), jnp.where(ok, hi, mid))

    lo, _ = jax.lax.fori_loop(
        0, 31, body, (jnp.int32(0), jnp.int32(0x7F800001)))
    gt = bits > lo
    cnt_gt = jnp.sum(jnp.where(gt, 1.0, 0.0))
    sum_gt = jnp.sum(jnp.where(gt, vals, 0.0))
    tval = jax.lax.bitcast_convert_type(lo, jnp.float32)
    hard_sum = sum_gt + (k - cnt_gt) * tval

    cls_neg = jnp.where(k > 0, hard_sum / jnp.maximum(k, 1.0), 0.0)
    cls_pos = jnp.where(npos > 0, pos_sum / jnp.maximum(npos, 1.0), 0.0)
    cls = jnp.where((nneg > 0) & (npos > 0), cls_pos + cls_neg, 0.0)
    reg = jnp.where(npos > 0, reg_sum / jnp.maximum(npos, 1.0), 0.0)
    total = cls + BETA * reg
    out_ref[...] = jnp.where(lane == 0, total,
                             jnp.where(lane == 1, cls, reg))


def kernel(y_pre, y_batch, anchor_boxes_xyxy, alpha):
    B, NA, C = y_pre.shape
    N = B * NA
    yp = y_pre.reshape(N, C)
    yb = y_batch.reshape(N, C)
    alpha_pad = jnp.zeros((8, 128), jnp.float32).at[0, :NUM_CLASSES].set(alpha)
    nab = NA // R

    neg, stats = pl.pallas_call(
        _stage_a,
        grid=(NBLK,),
        in_specs=[
            pl.BlockSpec((R, C), lambda i: (i, 0)),
            pl.BlockSpec((R, C), lambda i: (i, 0)),
            pl.BlockSpec((R, 4), lambda i: (i % nab, 0)),
            pl.BlockSpec((8, 128), lambda i: (0, 0)),
        ],
        out_specs=[
            pl.BlockSpec((R, 1), lambda i: (i, 0)),
            pl.BlockSpec((8, 128), lambda i: (0, 0)),
        ],
        out_shape=[
            jax.ShapeDtypeStruct((N, 1), jnp.float32),
            jax.ShapeDtypeStruct((8, 128), jnp.float32),
        ],
        scratch_shapes=[pltpu.SMEM((8,), jnp.float32)],
    )(yp, yb, anchor_boxes_xyxy, alpha_pad)

    out = pl.pallas_call(
        _stage_b,
        in_specs=[
            pl.BlockSpec((N // 128, 128), lambda: (0, 0)),
            pl.BlockSpec((8, 128), lambda: (0, 0)),
        ],
        out_specs=pl.BlockSpec((8, 128), lambda: (0, 0)),
        out_shape=jax.ShapeDtypeStruct((8, 128), jnp.float32),
    )(neg.reshape(N // 128, 128), stats)

    return out[0, 0], out[0, 1], out[0, 2]


# speedup vs baseline: 4.9320x; 4.9320x over previous
import jax
import jax.numpy as jnp
from jax.experimental import pallas as pl
from jax.experimental.pallas import tpu as pltpu

NC = 21
C = 25
GS = 50
BETA = 0.5
NEG_POS_RATIO = 3.0


def _stage_a(ypt_ref, ybt_ref, anc_ref, alpha_ref, neg_ref, stats_ref, acc_ref):
    i = pl.program_id(0)
    nsteps = pl.num_programs(0)

    @pl.when(i == 0)
    def _init():
        acc_ref[...] = jnp.zeros_like(acc_ref)

    cp = [ypt_ref[c, 0] for c in range(NC)]
    ch = [ybt_ref[c, 0] for c in range(NC)]

    m = cp[0]
    for c in range(1, NC):
        m = jnp.maximum(m, cp[c])
    se = jnp.exp(cp[0] - m)
    tl = ch[0] * cp[0]
    at = ch[0] * alpha_ref[0]
    for c in range(1, NC):
        se = se + jnp.exp(cp[c] - m)
        tl = tl + ch[c] * cp[c]
        at = at + ch[c] * alpha_ref[c]
    logpt = tl - m - jnp.log(se)
    pt = jnp.exp(logpt)
    omp = 1.0 - pt
    loss = -(omp * omp) * (logpt * at)

    negm = ch[0] > 0.5
    posm = jnp.logical_not(negm)

    neg_ref[0] = jnp.where(negm, loss + 0.0, -1.0)

    ax, ay, ax2, ay2 = (anc_ref[j, 0] for j in range(4))
    bpx, bpy, bpw, bph = (ypt_ref[NC + j, 0] for j in range(4))
    bhx, bhy, bhw, bhh = (ybt_ref[NC + j, 0] for j in range(4))
    wa = ax2 - ax
    ha = ay2 - ay
    cx = ax + 0.5 * wa
    cy = ay + 0.5 * ha
    dcxp = cx + bpx * wa
    dcyp = cy + bpy * ha
    dwp = wa * jnp.exp(bpw)
    dhp = ha * jnp.exp(bph)
    x1p = dcxp - 0.5 * dwp
    y1p = dcyp - 0.5 * dhp
    x2p = dcxp + 0.5 * dwp
    y2p = dcyp + 0.5 * dhp
    dcxt = cx + bhx * wa
    dcyt = cy + bhy * ha
    dwt = wa * jnp.exp(bhw)
    dht = ha * jnp.exp(bhh)
    x1t = dcxt - 0.5 * dwt
    y1t = dcyt - 0.5 * dht
    x2t = dcxt + 0.5 * dwt
    y2t = dcyt + 0.5 * dht
    iw = jnp.maximum(jnp.minimum(x2p, x2t) - jnp.maximum(x1p, x1t), 0.0)
    ih = jnp.maximum(jnp.minimum(y2p, y2t) - jnp.maximum(y1p, y1t), 0.0)
    inter = iw * ih
    a1 = jnp.maximum(x2p - x1p, 0.0) * jnp.maximum(y2p - y1p, 0.0)
    a2 = jnp.maximum(x2t - x1t, 0.0) * jnp.maximum(y2t - y1t, 0.0)
    union = a1 + a2 - inter
    iou = inter / (union + 1e-8)

    one = jnp.ones_like(loss)
    zero = jnp.zeros_like(loss)
    acc_ref[0] = acc_ref[0] + jnp.where(posm, loss, zero)
    acc_ref[1] = acc_ref[1] + jnp.where(posm, 1.0 - iou, zero)
    acc_ref[2] = acc_ref[2] + jnp.where(posm, one, zero)
    acc_ref[3] = acc_ref[3] + jnp.where(negm, one, zero)

    @pl.when(i == nsteps - 1)
    def _fin():
        lane = jax.lax.broadcasted_iota(jnp.int32, (8, 128), 1)
        v = jnp.where(lane == 0, jnp.sum(acc_ref[0]),
                      jnp.where(lane == 1, jnp.sum(acc_ref[1]),
                                jnp.where(lane == 2, jnp.sum(acc_ref[2]),
                                          jnp.sum(acc_ref[3]))))
        stats_ref[...] = v


def _stage_b(neg_ref, stats_ref, out_ref):
    lane = jax.lax.broadcasted_iota(jnp.int32, (8, 128), 1)
    row = jax.lax.broadcasted_iota(jnp.int32, (8, 128), 0)
    stats = stats_ref[...]
    sel = (row == 0)

    def pick(j):
        return jnp.sum(jnp.where(sel & (lane == j), stats, 0.0))

    pos_sum = pick(0)
    reg_sum = pick(1)
    npos = pick(2)
    nneg = pick(3)
    k = jnp.minimum(nneg, NEG_POS_RATIO * npos)

    vals = neg_ref[...]
    bits = jax.lax.bitcast_convert_type(vals, jnp.int32)

    def body(_, carry):
        lo, hi = carry
        mid = lo + (hi - lo) // 2
        cnt = jnp.sum(jnp.where(bits >= mid, 1.0, 0.0))
        ok = cnt >= k
        return (jnp.where(ok, mid, lo), jnp.where(ok, hi, mid))

    lo, _ = jax.lax.fori_loop(
        0, 31, body, (jnp.int32(0), jnp.int32(0x7F800001)))
    gt = bits > lo
    cnt_gt = jnp.sum(jnp.where(gt, 1.0, 0.0))
    sum_gt = jnp.sum(jnp.where(gt, vals, 0.0))
    tval = jax.lax.bitcast_convert_type(lo, jnp.float32)
    hard_sum = sum_gt + (k - cnt_gt) * tval

    cls_neg = jnp.where(k > 0, hard_sum / jnp.maximum(k, 1.0), 0.0)
    cls_pos = jnp.where(npos > 0, pos_sum / jnp.maximum(npos, 1.0), 0.0)
    cls = jnp.where((nneg > 0) & (npos > 0), cls_pos + cls_neg, 0.0)
    reg = jnp.where(npos > 0, reg_sum / jnp.maximum(npos, 1.0), 0.0)
    total = cls + BETA * reg
    out_ref[...] = jnp.where(lane == 0, total,
                             jnp.where(lane == 1, cls, reg))


def kernel(y_pre, y_batch, anchor_boxes_xyxy, alpha):
    B, NA, _ = y_pre.shape
    N = B * NA
    NL = N // 128
    grid = NL // GS

    ypt = jnp.transpose(y_pre.reshape(N, C)).reshape(C, grid, GS, 128)
    ybt = jnp.transpose(y_batch.reshape(N, C)).reshape(C, grid, GS, 128)
    anct = jnp.tile(jnp.transpose(anchor_boxes_xyxy),
                    (1, B)).reshape(4, grid, GS, 128)

    neg, stats = pl.pallas_call(
        _stage_a,
        grid=(grid,),
        in_specs=[
            pl.BlockSpec((C, 1, GS, 128), lambda i: (0, i, 0, 0)),
            pl.BlockSpec((C, 1, GS, 128), lambda i: (0, i, 0, 0)),
            pl.BlockSpec((4, 1, GS, 128), lambda i: (0, i, 0, 0)),
            pl.BlockSpec(memory_space=pltpu.SMEM),
        ],
        out_specs=[
            pl.BlockSpec((1, GS, 128), lambda i: (i, 0, 0)),
            pl.BlockSpec((8, 128), lambda i: (0, 0)),
        ],
        out_shape=[
            jax.ShapeDtypeStruct((grid, GS, 128), jnp.float32),
            jax.ShapeDtypeStruct((8, 128), jnp.float32),
        ],
        scratch_shapes=[pltpu.VMEM((4, GS, 128), jnp.float32)],
    )(ypt, ybt, anct, alpha)
    neg = neg.reshape(NL, 128)

    out = pl.pallas_call(
        _stage_b,
        in_specs=[
            pl.BlockSpec((NL, 128), lambda: (0, 0)),
            pl.BlockSpec((8, 128), lambda: (0, 0)),
        ],
        out_specs=pl.BlockSpec((8, 128), lambda: (0, 0)),
        out_shape=jax.ShapeDtypeStruct((8, 128), jnp.float32),
    )(neg, stats)

    return out[0, 0], out[0, 1], out[0, 2]
